# baseline (speedup 1.0000x reference)
import jax
import jax.numpy as jnp
from jax import lax
from jax.experimental import pallas as pl
from jax.experimental.pallas import tpu as pltpu

_MARGIN = 0.3
_NUM_CLASSES = 64


def _loss_kernel(x_ref, trow_ref, loss_ref, prec_ref):
    x = x_ref[...]
    t = trow_ref[...]
    n = x.shape[0]
    c_iota = lax.broadcasted_iota(jnp.int32, (_NUM_CLASSES, n), 0)
    pos = c_iota == t
    onehot = pos.astype(jnp.float32)
    counts = jnp.sum(onehot, axis=1, keepdims=True)
    class_sum = lax.dot_general(
        onehot, x, (((1,), (0,)), ((), ())),
        preferred_element_type=jnp.float32)
    centers = class_sum / jnp.maximum(counts, 1.0)
    ones_row = jnp.ones((1, x.shape[1]), jnp.float32)
    x_sq_row = lax.dot_general(
        ones_row, x * x, (((1,), (1,)), ((), ())),
        preferred_element_type=jnp.float32)
    g = lax.dot_general(
        centers, x, (((1,), (1,)), ((), ())),
        preferred_element_type=jnp.float32)
    d2 = x_sq_row - 2.0 * g
    m_pos = jnp.max(jnp.where(pos, d2, -jnp.inf), axis=1, keepdims=True)
    m_neg = jnp.min(jnp.where(pos, jnp.inf, d2), axis=1, keepdims=True)
    per_class = jnp.maximum(m_pos - m_neg + _MARGIN, 0.0)
    loss_ref[0, 0] = jnp.sum(counts * per_class) / n
    prec_ref[0, 0] = jnp.sum(
        counts * (m_neg > m_pos).astype(jnp.float32)) / n


def kernel(inputs, targets):
    t_row = targets.reshape(1, -1).astype(jnp.int32)
    loss, prec = pl.pallas_call(
        _loss_kernel,
        out_shape=(
            jax.ShapeDtypeStruct((1, 1), jnp.float32),
            jax.ShapeDtypeStruct((1, 1), jnp.float32),
        ),
        out_specs=(
            pl.BlockSpec(memory_space=pltpu.SMEM),
            pl.BlockSpec(memory_space=pltpu.SMEM),
        ),
    )(inputs, t_row)
    return loss[0, 0], prec[0, 0]


# speedup vs baseline: 1.0097x; 1.0097x over previous
import jax
import jax.numpy as jnp
from jax import lax
from jax.experimental import pallas as pl
from jax.experimental.pallas import tpu as pltpu

_MARGIN = 0.3
_NUM_CLASSES = 64


def _loss_kernel(x_ref, t_ref, loss_ref, prec_ref):
    x = x_ref[...]
    t = t_ref[...].reshape(1, -1)
    n = x.shape[0]
    c_iota = lax.broadcasted_iota(jnp.int32, (_NUM_CLASSES, n), 0)
    pos = c_iota == t
    onehot = pos.astype(jnp.float32)
    counts = jnp.sum(onehot, axis=1, keepdims=True)
    class_sum = lax.dot_general(
        onehot, x, (((1,), (0,)), ((), ())),
        preferred_element_type=jnp.float32)
    centers = class_sum / jnp.maximum(counts, 1.0)
    ones_row = jnp.ones((1, x.shape[1]), jnp.float32)
    x_sq_row = lax.dot_general(
        ones_row, x * x, (((1,), (1,)), ((), ())),
        preferred_element_type=jnp.float32)
    g = lax.dot_general(
        centers, x, (((1,), (1,)), ((), ())),
        preferred_element_type=jnp.float32)
    d2 = x_sq_row - 2.0 * g
    m_pos = jnp.max(jnp.where(pos, d2, -jnp.inf), axis=1, keepdims=True)
    m_neg = jnp.min(jnp.where(pos, jnp.inf, d2), axis=1, keepdims=True)
    per_class = jnp.maximum(m_pos - m_neg + _MARGIN, 0.0)
    loss_ref[0, 0] = jnp.sum(counts * per_class) / n
    prec_ref[0, 0] = jnp.sum(
        counts * (m_neg > m_pos).astype(jnp.float32)) / n


def kernel(inputs, targets):
    loss, prec = pl.pallas_call(
        _loss_kernel,
        out_shape=(
            jax.ShapeDtypeStruct((1, 1), jnp.float32),
            jax.ShapeDtypeStruct((1, 1), jnp.float32),
        ),
        out_specs=(
            pl.BlockSpec(memory_space=pltpu.SMEM),
            pl.BlockSpec(memory_space=pltpu.SMEM),
        ),
    )(inputs, targets)
    return loss[0, 0], prec[0, 0]
